# baseline (speedup 1.0000x reference)
import functools

import jax
import jax.numpy as jnp
from jax import lax
from jax.experimental import pallas as pl
from jax.experimental.pallas import tpu as pltpu
from jax.experimental.pallas import tpu_sc as plsc

N = 10000
N_PAD = 10240
E = 160000
NSUB = 16
EDGES_PER_TILE = 10240
E_PAD = NSUB * EDGES_PER_TILE
CHUNK = 128
NCHUNK = EDGES_PER_TILE // CHUNK
ROWS_PER_SUB = N_PAD // NSUB
D = 256
H = 128
BN = 2048


def _mesh():
    return plsc.VectorSubcoreMesh(core_axis_name="c", subcore_axis_name="s")


def _deg_body(edges_hbm, zeros_hbm, ones_hbm, out_hbm, idx_v, one_v, deg_sm):
    c = lax.axis_index("c")
    s = lax.axis_index("s")
    pltpu.sync_copy(edges_hbm.at[c].at[s], idx_v)
    pltpu.sync_copy(ones_hbm, one_v)

    base = pl.multiple_of(s * ROWS_PER_SUB, CHUNK)
    pltpu.sync_copy(zeros_hbm, deg_sm.at[pl.ds(base, ROWS_PER_SUB)])

    plsc.subcore_barrier()

    @pl.loop(0, NCHUNK)
    def _(j):
        pltpu.sync_copy(one_v, deg_sm.at[idx_v.at[j]], add=True)

    plsc.subcore_barrier()
    pltpu.sync_copy(deg_sm.at[pl.ds(base, ROWS_PER_SUB)], out_hbm.at[c].at[s])


_deg_kernel = functools.partial(
    pl.kernel,
    out_type=jax.ShapeDtypeStruct((2, NSUB, ROWS_PER_SUB, H), jnp.float32),
    mesh=_mesh(),
    scratch_types=[
        pltpu.VMEM((NCHUNK, CHUNK), jnp.int32),
        pltpu.VMEM((CHUNK, H), jnp.float32),
        pltpu.VMEM_SHARED((N_PAD, H), jnp.float32),
    ],
)(_deg_body)


def _agg_body(hs_hbm, src_hbm, dst_hbm, zeros_hbm, out_hbm,
              src_v, dst_v, rows0_v, rows1_v, agg_sm, gsem0, gsem1):
    c = lax.axis_index("c")
    s = lax.axis_index("s")

    base = pl.multiple_of(s * ROWS_PER_SUB, CHUNK)
    pltpu.sync_copy(zeros_hbm, agg_sm.at[pl.ds(base, ROWS_PER_SUB)])

    plsc.subcore_barrier()

    rows = (rows0_v, rows1_v)
    gsem = (gsem0, gsem1)
    HC = NCHUNK // 2

    for p in range(2):
        pltpu.sync_copy(src_hbm.at[c].at[p].at[s], src_v)
        pltpu.sync_copy(dst_hbm.at[p].at[s], dst_v)

        def gather_copy(j, b):
            e = pl.multiple_of(j * CHUNK, CHUNK)
            return pltpu.make_async_copy(
                hs_hbm.at[src_v.at[pl.ds(e, CHUNK)]], rows[b], gsem[b])

        gather_copy(0, 0).start()
        gather_copy(1, 1).start()

        @pl.loop(0, HC // 2)
        def _(g):
            for b in range(2):
                j = 2 * g + b
                gather_copy(j, b).wait()
                pltpu.sync_copy(rows[b], agg_sm.at[dst_v.at[j]], add=True)

                @pl.when(j + 2 < HC)
                def _():
                    gather_copy(j + 2, b).start()

    plsc.subcore_barrier()
    pltpu.sync_copy(agg_sm.at[pl.ds(base, ROWS_PER_SUB)], out_hbm.at[c].at[s])


_agg_kernel = functools.partial(
    pl.kernel,
    out_type=jax.ShapeDtypeStruct((2, NSUB, ROWS_PER_SUB, H), jnp.float32),
    mesh=_mesh(),
    scratch_types=[
        pltpu.VMEM((EDGES_PER_TILE // 2,), jnp.int32),
        pltpu.VMEM((NCHUNK // 2, CHUNK), jnp.int32),
        pltpu.VMEM((CHUNK, H), jnp.float32),
        pltpu.VMEM((CHUNK, H), jnp.float32),
        pltpu.VMEM_SHARED((N_PAD, H), jnp.float32),
        pltpu.SemaphoreType.DMA,
        pltpu.SemaphoreType.DMA,
    ],
)(_agg_body)


BM = 2000


def _tc_mm1_body(x_ref, w_ref, out_ref):
    hs = jnp.dot(x_ref[...], w_ref[...], preferred_element_type=jnp.float32)
    out_ref[0] = hs[:, :H]
    out_ref[1] = hs[:, H:]


_tc_mm1 = pl.pallas_call(
    _tc_mm1_body,
    grid=(N // BM,),
    in_specs=[
        pl.BlockSpec((BM, D), lambda i: (i, 0)),
        pl.BlockSpec((D, D), lambda i: (0, 0)),
    ],
    out_specs=pl.BlockSpec((2, BM, H), lambda i: (0, i, 0)),
    out_shape=jax.ShapeDtypeStruct((2, N, H), jnp.float32),
)


def _tc_scale_body(xw_ref, deg_ref, out_ref):
    nout = lax.rsqrt(jnp.maximum(deg_ref[:, 0:1], 1.0))
    out_ref[0] = xw_ref[0] * nout
    out_ref[1] = xw_ref[1] * nout


_tc_scale = pl.pallas_call(
    _tc_scale_body,
    grid=(N // BM,),
    in_specs=[
        pl.BlockSpec((2, BM, H), lambda i: (0, i, 0)),
        pl.BlockSpec((BM, H), lambda i: (i, 0)),
    ],
    out_specs=pl.BlockSpec((2, BM, H), lambda i: (0, i, 0)),
    out_shape=jax.ShapeDtypeStruct((2, N, H), jnp.float32),
)


def _tc_mid_body(agg_ref, degi_ref, dego_ref, b_ref, w_ref, out_ref):
    nin = lax.rsqrt(jnp.maximum(degi_ref[:, 0:1], 1.0))
    nout = lax.rsqrt(jnp.maximum(dego_ref[:, 0:1], 1.0))
    h0 = jnp.maximum(agg_ref[0] * nin + b_ref[0:1, :H], 0.0) * nout
    h1 = jnp.maximum(agg_ref[1] * nin + b_ref[0:1, H:], 0.0) * nout
    hs = (jnp.dot(h0, w_ref[:H, :], preferred_element_type=jnp.float32)
          + jnp.dot(h1, w_ref[H:, :], preferred_element_type=jnp.float32))
    out_ref[0] = hs[:, :H]
    out_ref[1] = hs[:, H:]


_tc_mid = pl.pallas_call(
    _tc_mid_body,
    grid=(N_PAD // BN,),
    in_specs=[
        pl.BlockSpec((2, BN, H), lambda i: (0, i, 0)),
        pl.BlockSpec((BN, H), lambda i: (i, 0)),
        pl.BlockSpec((BN, H), lambda i: (i, 0)),
        pl.BlockSpec((1, D), lambda i: (0, 0)),
        pl.BlockSpec((D, D), lambda i: (0, 0)),
    ],
    out_specs=pl.BlockSpec((2, BN, H), lambda i: (0, i, 0)),
    out_shape=jax.ShapeDtypeStruct((2, N_PAD, H), jnp.float32),
)


def _tc_last_body(agg_ref, degi_ref, b_ref, out_ref):
    nin = lax.rsqrt(jnp.maximum(degi_ref[:, 0:1], 1.0))
    out_ref[:, :H] = jnp.maximum(agg_ref[0] * nin + b_ref[0:1, :H], 0.0)
    out_ref[:, H:] = jnp.maximum(agg_ref[1] * nin + b_ref[0:1, H:], 0.0)


_tc_last = pl.pallas_call(
    _tc_last_body,
    grid=(N_PAD // BN,),
    in_specs=[
        pl.BlockSpec((2, BN, H), lambda i: (0, i, 0)),
        pl.BlockSpec((BN, H), lambda i: (i, 0)),
        pl.BlockSpec((1, D), lambda i: (0, 0)),
    ],
    out_specs=pl.BlockSpec((BN, D), lambda i: (i, 0)),
    out_shape=jax.ShapeDtypeStruct((N_PAD, D), jnp.float32),
)


@jax.jit
def _run(x, edge_index, W1, b1, W2, b2, W3, b3):
    src = edge_index[0]
    dst = edge_index[1]
    pad_e = E_PAD - E
    src_p = jnp.concatenate([src, jnp.full((pad_e,), N + 1, jnp.int32)])
    dst_p = jnp.concatenate([dst, jnp.full((pad_e,), N, jnp.int32)])
    src3 = src_p.reshape(NSUB, NCHUNK, CHUNK)
    dst3 = dst_p.reshape(NSUB, NCHUNK, CHUNK)
    edges2 = jnp.stack([src3, dst3])
    srcps = src_p.reshape(NSUB, 2, EDGES_PER_TILE // 2).swapaxes(0, 1)
    srcps = jnp.stack([srcps, srcps + N_PAD])
    dstps = dst_p.reshape(NSUB, 2, NCHUNK // 2, CHUNK).swapaxes(0, 1)
    zerosH = jnp.zeros((ROWS_PER_SUB, H), jnp.float32)
    onesH = jnp.ones((CHUNK, H), jnp.float32)

    degs = _deg_kernel(edges2, zerosH, onesH).reshape(2, N_PAD, H)
    deg_out = degs[0]
    deg_in = degs[1]

    b1r = b1.reshape(1, D)
    b2r = b2.reshape(1, D)
    b3r = b3.reshape(1, D)

    xw1 = _tc_mm1(x, W1)
    hs1 = jnp.pad(_tc_scale(xw1, deg_out[:N]), ((0, 0), (0, N_PAD - N), (0, 0)))
    agg1 = _agg_kernel(hs1.reshape(2 * N_PAD, H), srcps, dstps, zerosH).reshape(2, N_PAD, H)
    hs2 = _tc_mid(agg1, deg_in, deg_out, b1r, W2)
    agg2 = _agg_kernel(hs2.reshape(2 * N_PAD, H), srcps, dstps, zerosH).reshape(2, N_PAD, H)
    hs3 = _tc_mid(agg2, deg_in, deg_out, b2r, W3)
    agg3 = _agg_kernel(hs3.reshape(2 * N_PAD, H), srcps, dstps, zerosH).reshape(2, N_PAD, H)
    return _tc_last(agg3, deg_in, b3r)[:N]


def kernel(x, edge_index, W1, b1, W2, b2, W3, b3):
    return _run(x, edge_index, W1, b1, W2, b2, W3, b3)


# speedup vs baseline: 1.0331x; 1.0331x over previous
import functools

import jax
import jax.numpy as jnp
from jax import lax
from jax.experimental import pallas as pl
from jax.experimental.pallas import tpu as pltpu
from jax.experimental.pallas import tpu_sc as plsc

N = 10000
N_PAD = 10240
E = 160000
NSUB = 16
EDGES_PER_TILE = 10240
E_PAD = NSUB * EDGES_PER_TILE
CHUNK = 128
NCHUNK = EDGES_PER_TILE // CHUNK
ROWS_PER_SUB = N_PAD // NSUB
D = 256
H = 128
BN = 2048


def _mesh():
    return plsc.VectorSubcoreMesh(core_axis_name="c", subcore_axis_name="s")


def _deg_body(edges_hbm, zeros_hbm, ones_hbm, out_hbm, idx_v, one_v, deg_sm):
    c = lax.axis_index("c")
    s = lax.axis_index("s")
    pltpu.sync_copy(edges_hbm.at[c].at[s], idx_v)
    pltpu.sync_copy(ones_hbm, one_v)

    base = pl.multiple_of(s * ROWS_PER_SUB, CHUNK)
    pltpu.sync_copy(zeros_hbm, deg_sm.at[pl.ds(base, ROWS_PER_SUB)])

    plsc.subcore_barrier()

    @pl.loop(0, NCHUNK)
    def _(j):
        pltpu.sync_copy(one_v, deg_sm.at[idx_v.at[j]], add=True)

    plsc.subcore_barrier()
    pltpu.sync_copy(deg_sm.at[pl.ds(base, ROWS_PER_SUB)], out_hbm.at[c].at[s])


_deg_kernel = functools.partial(
    pl.kernel,
    out_type=jax.ShapeDtypeStruct((2, NSUB, ROWS_PER_SUB, H), jnp.float32),
    mesh=_mesh(),
    scratch_types=[
        pltpu.VMEM((NCHUNK, CHUNK), jnp.int32),
        pltpu.VMEM((CHUNK, H), jnp.float32),
        pltpu.VMEM_SHARED((N_PAD, H), jnp.float32),
    ],
)(_deg_body)


def _agg_body(hs_hbm, src_hbm, dst_hbm, zeros_hbm, out_hbm,
              src_v, dst_v, rows0_v, rows1_v, agg_sm, gsem0, gsem1):
    c = lax.axis_index("c")
    s = lax.axis_index("s")

    base = pl.multiple_of(s * ROWS_PER_SUB, CHUNK)
    pltpu.sync_copy(zeros_hbm, agg_sm.at[pl.ds(base, ROWS_PER_SUB)])

    plsc.subcore_barrier()

    rows = (rows0_v, rows1_v)
    gsem = (gsem0, gsem1)
    HC = NCHUNK // 2

    for p in range(2):
        pltpu.sync_copy(src_hbm.at[c].at[p].at[s], src_v)
        pltpu.sync_copy(dst_hbm.at[p].at[s], dst_v)

        def gather_copy(j, b):
            e = pl.multiple_of(j * CHUNK, CHUNK)
            return pltpu.make_async_copy(
                hs_hbm.at[src_v.at[pl.ds(e, CHUNK)]], rows[b], gsem[b])

        gather_copy(0, 0).start()
        gather_copy(1, 1).start()

        @pl.loop(0, HC // 2)
        def _(g):
            for b in range(2):
                j = 2 * g + b
                gather_copy(j, b).wait()
                pltpu.sync_copy(rows[b], agg_sm.at[dst_v.at[j]], add=True)

                @pl.when(j + 2 < HC)
                def _():
                    gather_copy(j + 2, b).start()

    plsc.subcore_barrier()
    pltpu.sync_copy(agg_sm.at[pl.ds(base, ROWS_PER_SUB)], out_hbm.at[c].at[s])


_agg_kernel = functools.partial(
    pl.kernel,
    out_type=jax.ShapeDtypeStruct((2, NSUB, ROWS_PER_SUB, H), jnp.float32),
    mesh=_mesh(),
    scratch_types=[
        pltpu.VMEM((EDGES_PER_TILE // 2,), jnp.int32),
        pltpu.VMEM((NCHUNK // 2, CHUNK), jnp.int32),
        pltpu.VMEM((CHUNK, H), jnp.float32),
        pltpu.VMEM((CHUNK, H), jnp.float32),
        pltpu.VMEM_SHARED((N_PAD, H), jnp.float32),
        pltpu.SemaphoreType.DMA,
        pltpu.SemaphoreType.DMA,
    ],
)(_agg_body)


BM = 2000


def _tc_mm1_body(x_ref, w_ref, out_ref):
    hs = jnp.dot(x_ref[...], w_ref[...], preferred_element_type=jnp.float32)
    out_ref[0] = hs[:, :H]
    out_ref[1] = hs[:, H:]


_tc_mm1 = pl.pallas_call(
    _tc_mm1_body,
    grid=(N // BM,),
    in_specs=[
        pl.BlockSpec((BM, D), lambda i: (i, 0)),
        pl.BlockSpec((D, D), lambda i: (0, 0)),
    ],
    out_specs=pl.BlockSpec((2, BM, H), lambda i: (0, i, 0)),
    out_shape=jax.ShapeDtypeStruct((2, N, H), jnp.float32),
)


def _tc_scale_body(xw_ref, deg_ref, out_ref):
    nout = lax.rsqrt(jnp.maximum(deg_ref[:, 0:1], 1.0))
    out_ref[0] = xw_ref[0] * nout
    out_ref[1] = xw_ref[1] * nout


_tc_scale = pl.pallas_call(
    _tc_scale_body,
    grid=(N // BM,),
    in_specs=[
        pl.BlockSpec((2, BM, H), lambda i: (0, i, 0)),
        pl.BlockSpec((BM, H), lambda i: (i, 0)),
    ],
    out_specs=pl.BlockSpec((2, BM, H), lambda i: (0, i, 0)),
    out_shape=jax.ShapeDtypeStruct((2, N_PAD, H), jnp.float32),
)


def _tc_mid_body(agg_ref, degi_ref, dego_ref, b_ref, w_ref, out_ref):
    nin = lax.rsqrt(jnp.maximum(degi_ref[:, 0:1], 1.0))
    nout = lax.rsqrt(jnp.maximum(dego_ref[:, 0:1], 1.0))
    h0 = jnp.maximum(agg_ref[0] * nin + b_ref[0:1, :H], 0.0) * nout
    h1 = jnp.maximum(agg_ref[1] * nin + b_ref[0:1, H:], 0.0) * nout
    hs = (jnp.dot(h0, w_ref[:H, :], preferred_element_type=jnp.float32)
          + jnp.dot(h1, w_ref[H:, :], preferred_element_type=jnp.float32))
    out_ref[0] = hs[:, :H]
    out_ref[1] = hs[:, H:]


_tc_mid = pl.pallas_call(
    _tc_mid_body,
    grid=(N_PAD // BN,),
    in_specs=[
        pl.BlockSpec((2, BN, H), lambda i: (0, i, 0)),
        pl.BlockSpec((BN, H), lambda i: (i, 0)),
        pl.BlockSpec((BN, H), lambda i: (i, 0)),
        pl.BlockSpec((1, D), lambda i: (0, 0)),
        pl.BlockSpec((D, D), lambda i: (0, 0)),
    ],
    out_specs=pl.BlockSpec((2, BN, H), lambda i: (0, i, 0)),
    out_shape=jax.ShapeDtypeStruct((2, N_PAD, H), jnp.float32),
)


def _tc_last_body(agg_ref, degi_ref, b_ref, out_ref):
    nin = lax.rsqrt(jnp.maximum(degi_ref[:, 0:1], 1.0))
    out_ref[:, :H] = jnp.maximum(agg_ref[0] * nin + b_ref[0:1, :H], 0.0)
    out_ref[:, H:] = jnp.maximum(agg_ref[1] * nin + b_ref[0:1, H:], 0.0)


_tc_last = pl.pallas_call(
    _tc_last_body,
    grid=(N_PAD // BN,),
    in_specs=[
        pl.BlockSpec((2, BN, H), lambda i: (0, i, 0)),
        pl.BlockSpec((BN, H), lambda i: (i, 0)),
        pl.BlockSpec((1, D), lambda i: (0, 0)),
    ],
    out_specs=pl.BlockSpec((BN, D), lambda i: (i, 0)),
    out_shape=jax.ShapeDtypeStruct((N_PAD, D), jnp.float32),
)


@jax.jit
def _run(x, edge_index, W1, b1, W2, b2, W3, b3):
    src = edge_index[0]
    dst = edge_index[1]
    pad_e = E_PAD - E
    src_p = jnp.concatenate([src, jnp.full((pad_e,), N + 1, jnp.int32)])
    dst_p = jnp.concatenate([dst, jnp.full((pad_e,), N, jnp.int32)])
    src3 = src_p.reshape(NSUB, NCHUNK, CHUNK)
    dst3 = dst_p.reshape(NSUB, NCHUNK, CHUNK)
    edges2 = jnp.stack([src3, dst3])
    srcps = src_p.reshape(NSUB, 2, EDGES_PER_TILE // 2).swapaxes(0, 1)
    srcps = jnp.stack([srcps, srcps + N_PAD])
    dstps = dst_p.reshape(NSUB, 2, NCHUNK // 2, CHUNK).swapaxes(0, 1)
    zerosH = jnp.zeros((ROWS_PER_SUB, H), jnp.float32)
    onesH = jnp.ones((CHUNK, H), jnp.float32)

    degs = _deg_kernel(edges2, zerosH, onesH).reshape(2, N_PAD, H)
    deg_out = degs[0]
    deg_in = degs[1]

    b1r = b1.reshape(1, D)
    b2r = b2.reshape(1, D)
    b3r = b3.reshape(1, D)

    xw1 = _tc_mm1(x, W1)
    hs1 = _tc_scale(xw1, deg_out[:N])
    agg1 = _agg_kernel(hs1.reshape(2 * N_PAD, H), srcps, dstps, zerosH).reshape(2, N_PAD, H)
    hs2 = _tc_mid(agg1, deg_in, deg_out, b1r, W2)
    agg2 = _agg_kernel(hs2.reshape(2 * N_PAD, H), srcps, dstps, zerosH).reshape(2, N_PAD, H)
    hs3 = _tc_mid(agg2, deg_in, deg_out, b2r, W3)
    agg3 = _agg_kernel(hs3.reshape(2 * N_PAD, H), srcps, dstps, zerosH).reshape(2, N_PAD, H)
    return _tc_last(agg3, deg_in, b3r)[:N]


def kernel(x, edge_index, W1, b1, W2, b2, W3, b3):
    return _run(x, edge_index, W1, b1, W2, b2, W3, b3)
